# baseline (speedup 1.0000x reference)
import functools

import jax
import jax.numpy as jnp
from jax import lax
from jax.experimental import pallas as pl
from jax.experimental.pallas import tpu as pltpu
from jax.experimental.pallas import tpu_sc as plsc

NUM_WORKERS = 32
N = 800000
D = 64
RPW = N // NUM_WORKERS
CR = 360
NFULL = RPW // CR
TCR = RPW - NFULL * CR


def _sc_lookup(edge_type, table_flat):
    mesh = plsc.VectorSubcoreMesh(core_axis_name="c", subcore_axis_name="s")

    @functools.partial(
        pl.kernel,
        mesh=mesh,
        out_type=jax.ShapeDtypeStruct((N, D), jnp.float32),
        scratch_types=[
            pltpu.VMEM((3 * D,), jnp.float32),
            pltpu.VMEM((RPW,), jnp.int32),
            pltpu.VMEM((CR, D), jnp.float32),
            pltpu.VMEM((CR, D), jnp.float32),
            pltpu.SemaphoreType.DMA,
            pltpu.SemaphoreType.DMA,
        ],
    )
    def body(idx_hbm, tab_hbm, out_hbm, tab_v, idx_v, rows_a, rows_b,
             sem_a, sem_b):
        wid = lax.axis_index("s") * 2 + lax.axis_index("c")
        base = wid * RPW
        pltpu.sync_copy(tab_hbm, tab_v)
        pltpu.sync_copy(idx_hbm.at[pl.ds(base, RPW)], idx_v)

        def copy_rows(rows_v, off, b, njs):
            v = idx_v[pl.ds(off + 16 * b, 16)] * D
            for j in range(njs):
                s = v[j]
                r = 16 * b + j
                for k in range(D // 16):
                    rows_v[r, pl.ds(16 * k, 16)] = (
                        tab_v[pl.ds(s + 16 * k, 16)])

        def build(m, nrows, rows_v, sem):
            off = m * CR
            nb, tail = nrows // 16, nrows % 16

            def row_block(b, carry):
                copy_rows(rows_v, off, b, 16)
                return carry

            lax.fori_loop(0, nb, row_block, 0)
            if tail:
                copy_rows(rows_v, off, nb, tail)
            pltpu.async_copy(rows_v.at[pl.ds(0, nrows)],
                             out_hbm.at[pl.ds(base + off, nrows)], sem)

        def drain(m, nrows, rows_v, sem):
            pltpu.make_async_copy(
                rows_v.at[pl.ds(0, nrows)],
                out_hbm.at[pl.ds(base + m * CR, nrows)], sem).wait()

        build(0, CR, rows_a, sem_a)

        def step(k, carry):
            m1 = 2 * k + 1
            build(m1, CR, rows_b, sem_b)
            drain(m1 - 1, CR, rows_a, sem_a)
            build(m1 + 1, CR, rows_a, sem_a)
            drain(m1, CR, rows_b, sem_b)
            return carry

        lax.fori_loop(0, (NFULL - 1) // 2, step, 0)
        build(NFULL, TCR, rows_b, sem_b)
        drain(NFULL - 1, CR, rows_a, sem_a)
        drain(NFULL, TCR, rows_b, sem_b)

    return body(edge_type, table_flat)


def kernel(edge_type, table):
    table_flat = table.astype(jnp.float32).reshape(3 * D)
    return _sc_lookup(edge_type.astype(jnp.int32), table_flat)


# speedup vs baseline: 1.5604x; 1.5604x over previous
import functools

import jax
import jax.numpy as jnp
from jax import lax
from jax.experimental import pallas as pl
from jax.experimental.pallas import tpu as pltpu
from jax.experimental.pallas import tpu_sc as plsc

NUM_WORKERS = 32
N = 800000
D = 64
RPW = N // NUM_WORKERS
CR = 360
NFULL = RPW // CR
TCR = RPW - NFULL * CR


def _sc_lookup(edge_type, table_flat):
    mesh = plsc.VectorSubcoreMesh(core_axis_name="c", subcore_axis_name="s")

    @functools.partial(
        pl.kernel,
        mesh=mesh,
        out_type=jax.ShapeDtypeStruct((N, D), jnp.float32),
        scratch_types=[
            pltpu.VMEM((3 * D,), jnp.float32),
            pltpu.VMEM((RPW,), jnp.int32),
            pltpu.VMEM((CR, D), jnp.float32),
            pltpu.VMEM((CR, D), jnp.float32),
            pltpu.SemaphoreType.DMA,
            pltpu.SemaphoreType.DMA,
        ],
    )
    def body(idx_hbm, tab_hbm, out_hbm, tab_v, idx_v, rows_a, rows_b,
             sem_a, sem_b):
        wid = lax.axis_index("s") * 2 + lax.axis_index("c")
        base = wid * RPW
        pltpu.sync_copy(tab_hbm, tab_v)
        pltpu.sync_copy(idx_hbm.at[pl.ds(base, RPW)], idx_v)

        T = [[tab_v[pl.ds(64 * s + 16 * k, 16)] for k in range(D // 16)]
             for s in range(3)]
        L = [T[1][k] - T[0][k] for k in range(D // 16)]
        Q = [T[2][k] - T[1][k] - L[k] for k in range(D // 16)]
        dnums = lax.GatherDimensionNumbers(
            offset_dims=(), collapsed_slice_dims=(0,), start_index_map=(0,))
        splats = [jnp.full((16, 1), j, jnp.int32) for j in range(16)]

        def copy_rows(rows_v, off, b, njs):
            v = idx_v[pl.ds(off + 16 * b, 16)]
            for j in range(njs):
                sj = lax.gather(v, splats[j], dnums, slice_sizes=(1,),
                                mode=lax.GatherScatterMode.PROMISE_IN_BOUNDS)
                sjf = sj.astype(jnp.float32)
                q = sjf * (sjf - 1.0) * 0.5
                r = 16 * b + j
                for k in range(D // 16):
                    rows_v[r, pl.ds(16 * k, 16)] = (
                        T[0][k] + sjf * L[k] + q * Q[k])

        def build(m, nrows, rows_v, sem):
            off = m * CR
            nb, tail = nrows // 16, nrows % 16

            def row_block(b, carry):
                copy_rows(rows_v, off, b, 16)
                return carry

            lax.fori_loop(0, nb, row_block, 0)
            if tail:
                copy_rows(rows_v, off, nb, tail)
            pltpu.async_copy(rows_v.at[pl.ds(0, nrows)],
                             out_hbm.at[pl.ds(base + off, nrows)], sem)

        def drain(m, nrows, rows_v, sem):
            pltpu.make_async_copy(
                rows_v.at[pl.ds(0, nrows)],
                out_hbm.at[pl.ds(base + m * CR, nrows)], sem).wait()

        build(0, CR, rows_a, sem_a)

        def step(k, carry):
            m1 = 2 * k + 1
            build(m1, CR, rows_b, sem_b)
            drain(m1 - 1, CR, rows_a, sem_a)
            build(m1 + 1, CR, rows_a, sem_a)
            drain(m1, CR, rows_b, sem_b)
            return carry

        lax.fori_loop(0, (NFULL - 1) // 2, step, 0)
        build(NFULL, TCR, rows_b, sem_b)
        drain(NFULL - 1, CR, rows_a, sem_a)
        drain(NFULL, TCR, rows_b, sem_b)

    return body(edge_type, table_flat)


def kernel(edge_type, table):
    table_flat = table.astype(jnp.float32).reshape(3 * D)
    return _sc_lookup(edge_type.astype(jnp.int32), table_flat)
